# Initial kernel scaffold; baseline (speedup 1.0000x reference)
import functools

import jax
import jax.numpy as jnp
from jax import lax
from jax.experimental import pallas as pl
from jax.experimental.pallas import tpu as pltpu

_MARGIN = 0.1
_NNEG = 5
_TRIPLET = _NNEG + 2


def _body(nq, l, nchunk, chunk, nlabel,
          q_ref, rows_ref, trow_ref, qidx_ref, qidxs_ref, pidxf_ref, nnegf_ref,
          out_ref,
          pidx_s, nneg_s, hh_s, posval_s, negval_s, pmax_s, nmax_s, pcnt_s):
    c = pl.program_id(0)

    @pl.when(c == 0)
    def _init():
        qv = qidxs_ref[0:1, :]
        qi = qidx_ref[...]
        eq = qi == qv
        iota1 = lax.broadcasted_iota(jnp.int32, (nq, l), 1)
        qloc = jnp.min(jnp.where(eq, iota1, l), axis=1, keepdims=True)
        hh = qloc < l
        onehot = jnp.where((iota1 == qloc) & hh, 1.0, 0.0).astype(jnp.float32)
        pidx_s[...] = lax.dot(onehot, pidxf_ref[...],
                              preferred_element_type=jnp.float32).astype(jnp.int32)
        nneg_s[...] = lax.dot(onehot, nnegf_ref[...],
                              preferred_element_type=jnp.float32).astype(jnp.int32)
        hh_s[...] = hh.astype(jnp.int32)
        pmax_s[...] = jnp.full((nq, 1), -jnp.inf, jnp.float32)
        nmax_s[...] = jnp.full((nq, 1), -jnp.inf, jnp.float32)
        pcnt_s[...] = jnp.zeros((nq, 1), jnp.float32)

    rows = rows_ref[...]
    sim = lax.dot_general(q_ref[...], rows, (((1,), (1,)), ((), ())),
                          preferred_element_type=jnp.float32)
    trow = trow_ref[0]
    pidx = pidx_s[...]
    nneg = nneg_s[...]
    pos = trow == pidx[:, 0:1]
    negin = trow == nneg[:, 0:1]
    for j in range(1, nlabel):
        pos = pos | (trow == pidx[:, j:j + 1])
        negin = negin | (trow == nneg[:, j:j + 1])
    neg = ~negin

    posv = jnp.where(pos, sim, jnp.inf)
    negv = jnp.where(neg, sim, -jnp.inf)
    posval_s[:, pl.ds(c * chunk, chunk)] = posv
    negval_s[:, pl.ds(c * chunk, chunk)] = negv
    pmax_s[...] = jnp.maximum(
        pmax_s[...],
        jnp.max(jnp.where(pos, sim, -jnp.inf), axis=1, keepdims=True))
    nmax_s[...] = jnp.maximum(nmax_s[...], jnp.max(negv, axis=1, keepdims=True))
    pcnt_s[...] += jnp.sum(pos.astype(jnp.float32), axis=1, keepdims=True)

    @pl.when(c == nchunk - 1)
    def _finale():
        pmax = pmax_s[...]
        nmax = nmax_s[...]
        pcnt = pcnt_s[...]
        pt = nmax + _MARGIN
        nt = jnp.maximum(0.4, pmax) - _MARGIN
        zero = jnp.zeros((nq, 1), jnp.float32)
        pos_n = zero
        pos_sum = zero
        neg_n = zero
        neg_sum = zero
        for k in range(nchunk):
            pv = posval_s[:, k * chunk:(k + 1) * chunk]
            nv = negval_s[:, k * chunk:(k + 1) * chunk]
            selp = pv < pt
            seln = nv > nt
            pos_n = pos_n + jnp.sum(selp.astype(jnp.float32), axis=1, keepdims=True)
            pos_sum = pos_sum + jnp.sum(jnp.where(selp, 1.0 - pv, 0.0), axis=1,
                                        keepdims=True)
            neg_n = neg_n + jnp.sum(seln.astype(jnp.float32), axis=1, keepdims=True)
            neg_sum = neg_sum + jnp.sum(jnp.where(seln, nv, 0.0), axis=1,
                                        keepdims=True)
        pos_loss = jnp.where(pos_n > 0, pos_sum / jnp.maximum(pos_n, 1.0), 0.0)
        neg_loss = jnp.where(neg_n > 0, neg_sum / jnp.maximum(neg_n, 1.0), 0.0)
        contrib = jnp.where((hh_s[...] > 0) & (pcnt > 0), pos_loss + neg_loss, 0.0)
        out_ref[...] = (jnp.sum(contrib) / nq).reshape(1, 1)


@jax.jit
def kernel(inputs_col, targets_col, inputs_row, targets_row, qidxs, pidxs, nnegs):
    n, d = inputs_col.shape
    nrow = inputs_row.shape[0]
    l = qidxs.shape[0]
    nlabel = pidxs.shape[1]
    nq = n // _TRIPLET

    chunk = 4096
    nchunk = nrow // chunk

    q = inputs_col[::_TRIPLET]
    qidx = targets_col[::_TRIPLET].reshape(nq, 1)
    qidxs2 = jnp.broadcast_to(qidxs[None, :], (8, l))
    trow3 = targets_row.reshape(nchunk, 1, chunk)
    pidxf = pidxs.astype(jnp.float32)
    nnegf = nnegs.astype(jnp.float32)

    out = pl.pallas_call(
        functools.partial(_body, nq, l, nchunk, chunk, nlabel),
        grid=(nchunk,),
        in_specs=[
            pl.BlockSpec((nq, d), lambda c: (0, 0)),
            pl.BlockSpec((chunk, d), lambda c: (c, 0)),
            pl.BlockSpec((1, 1, chunk), lambda c: (c, 0, 0)),
            pl.BlockSpec((nq, 1), lambda c: (0, 0)),
            pl.BlockSpec((8, l), lambda c: (0, 0)),
            pl.BlockSpec((l, nlabel), lambda c: (0, 0)),
            pl.BlockSpec((l, nlabel), lambda c: (0, 0)),
        ],
        out_specs=pl.BlockSpec((1, 1), lambda c: (0, 0)),
        out_shape=jax.ShapeDtypeStruct((1, 1), jnp.float32),
        scratch_shapes=[
            pltpu.VMEM((nq, nlabel), jnp.int32),
            pltpu.VMEM((nq, nlabel), jnp.int32),
            pltpu.VMEM((nq, 1), jnp.int32),
            pltpu.VMEM((nq, nrow), jnp.float32),
            pltpu.VMEM((nq, nrow), jnp.float32),
            pltpu.VMEM((nq, 1), jnp.float32),
            pltpu.VMEM((nq, 1), jnp.float32),
            pltpu.VMEM((nq, 1), jnp.float32),
        ],
    )(q, inputs_row, trow3, qidx, qidxs2, pidxf, nnegf)
    return out.reshape(1)


# speedup vs baseline: 39.1336x; 39.1336x over previous
import functools

import jax
import jax.numpy as jnp
from jax import lax
from jax.experimental import pallas as pl
from jax.experimental.pallas import tpu as pltpu

_MARGIN = 0.1
_NNEG = 5
_TRIPLET = _NNEG + 2


def _body(nq, l, nchunk, chunk, nlabel,
          q_ref, rows_ref, trow_ref, qidx_ref, qidxs_ref, pidxf_ref, nnegf_ref,
          out_ref,
          pidx_s, nneg_s, hh_s, posval_s, negval_s, pmax_s, nmax_s, pcnt_s):
    c = pl.program_id(0)

    @pl.when(c == 0)
    def _init():
        qv = qidxs_ref[0:1, :]
        qi = qidx_ref[...]
        eq = qi == qv
        iota1 = lax.broadcasted_iota(jnp.int32, (nq, l), 1)
        qloc = jnp.min(jnp.where(eq, iota1, l), axis=1, keepdims=True)
        hh = qloc < l
        onehot = jnp.where((iota1 == qloc) & hh, 1.0, 0.0).astype(jnp.float32)
        pidx_s[...] = lax.dot(onehot, pidxf_ref[...],
                              precision=lax.Precision.HIGHEST,
                              preferred_element_type=jnp.float32).astype(jnp.int32)
        nneg_s[...] = lax.dot(onehot, nnegf_ref[...],
                              precision=lax.Precision.HIGHEST,
                              preferred_element_type=jnp.float32).astype(jnp.int32)
        hh_s[...] = hh.astype(jnp.int32)
        pmax_s[...] = jnp.full((nq, 1), -jnp.inf, jnp.float32)
        nmax_s[...] = jnp.full((nq, 1), -jnp.inf, jnp.float32)
        pcnt_s[...] = jnp.zeros((nq, 1), jnp.float32)

    rows = rows_ref[...]
    sim = lax.dot_general(q_ref[...], rows, (((1,), (1,)), ((), ())),
                          precision=lax.Precision.HIGHEST,
                          preferred_element_type=jnp.float32)
    trow = trow_ref[0]
    pidx = pidx_s[...]
    nneg = nneg_s[...]
    pos = trow == pidx[:, 0:1]
    negin = trow == nneg[:, 0:1]
    for j in range(1, nlabel):
        pos = pos | (trow == pidx[:, j:j + 1])
        negin = negin | (trow == nneg[:, j:j + 1])
    neg = ~negin

    posv = jnp.where(pos, sim, jnp.inf)
    negv = jnp.where(neg, sim, -jnp.inf)
    posval_s[:, pl.ds(c * chunk, chunk)] = posv
    negval_s[:, pl.ds(c * chunk, chunk)] = negv
    pmax_s[...] = jnp.maximum(
        pmax_s[...],
        jnp.max(jnp.where(pos, sim, -jnp.inf), axis=1, keepdims=True))
    nmax_s[...] = jnp.maximum(nmax_s[...], jnp.max(negv, axis=1, keepdims=True))
    pcnt_s[...] += jnp.sum(pos.astype(jnp.float32), axis=1, keepdims=True)

    @pl.when(c == nchunk - 1)
    def _finale():
        pmax = pmax_s[...]
        nmax = nmax_s[...]
        pcnt = pcnt_s[...]
        pt = nmax + _MARGIN
        nt = jnp.maximum(0.4, pmax) - _MARGIN
        zero = jnp.zeros((nq, 1), jnp.float32)
        pos_n = zero
        pos_sum = zero
        neg_n = zero
        neg_sum = zero
        for k in range(nchunk):
            pv = posval_s[:, k * chunk:(k + 1) * chunk]
            nv = negval_s[:, k * chunk:(k + 1) * chunk]
            selp = pv < pt
            seln = nv > nt
            pos_n = pos_n + jnp.sum(selp.astype(jnp.float32), axis=1, keepdims=True)
            pos_sum = pos_sum + jnp.sum(jnp.where(selp, 1.0 - pv, 0.0), axis=1,
                                        keepdims=True)
            neg_n = neg_n + jnp.sum(seln.astype(jnp.float32), axis=1, keepdims=True)
            neg_sum = neg_sum + jnp.sum(jnp.where(seln, nv, 0.0), axis=1,
                                        keepdims=True)
        pos_loss = jnp.where(pos_n > 0, pos_sum / jnp.maximum(pos_n, 1.0), 0.0)
        neg_loss = jnp.where(neg_n > 0, neg_sum / jnp.maximum(neg_n, 1.0), 0.0)
        contrib = jnp.where((hh_s[...] > 0) & (pcnt > 0), pos_loss + neg_loss, 0.0)
        out_ref[...] = (jnp.sum(contrib) / nq).reshape(1, 1)


@jax.jit
def kernel(inputs_col, targets_col, inputs_row, targets_row, qidxs, pidxs, nnegs):
    n, d = inputs_col.shape
    nrow = inputs_row.shape[0]
    l = qidxs.shape[0]
    nlabel = pidxs.shape[1]
    nq = n // _TRIPLET

    chunk = 4096
    nchunk = nrow // chunk

    q = inputs_col[::_TRIPLET]
    qidx = targets_col[::_TRIPLET].reshape(nq, 1)
    qidxs2 = jnp.broadcast_to(qidxs[None, :], (8, l))
    trow3 = targets_row.reshape(nchunk, 1, chunk)
    pidxf = pidxs.astype(jnp.float32)
    nnegf = nnegs.astype(jnp.float32)

    out = pl.pallas_call(
        functools.partial(_body, nq, l, nchunk, chunk, nlabel),
        grid=(nchunk,),
        in_specs=[
            pl.BlockSpec((nq, d), lambda c: (0, 0)),
            pl.BlockSpec((chunk, d), lambda c: (c, 0)),
            pl.BlockSpec((1, 1, chunk), lambda c: (c, 0, 0)),
            pl.BlockSpec((nq, 1), lambda c: (0, 0)),
            pl.BlockSpec((8, l), lambda c: (0, 0)),
            pl.BlockSpec((l, nlabel), lambda c: (0, 0)),
            pl.BlockSpec((l, nlabel), lambda c: (0, 0)),
        ],
        out_specs=pl.BlockSpec((1, 1), lambda c: (0, 0)),
        out_shape=jax.ShapeDtypeStruct((1, 1), jnp.float32),
        scratch_shapes=[
            pltpu.VMEM((nq, nlabel), jnp.int32),
            pltpu.VMEM((nq, nlabel), jnp.int32),
            pltpu.VMEM((nq, 1), jnp.int32),
            pltpu.VMEM((nq, nrow), jnp.float32),
            pltpu.VMEM((nq, nrow), jnp.float32),
            pltpu.VMEM((nq, 1), jnp.float32),
            pltpu.VMEM((nq, 1), jnp.float32),
            pltpu.VMEM((nq, 1), jnp.float32),
        ],
    )(q, inputs_row, trow3, qidx, qidxs2, pidxf, nnegf)
    return out.reshape(1)


# speedup vs baseline: 46.7243x; 1.1940x over previous
import functools

import jax
import jax.numpy as jnp
from jax import lax
from jax.experimental import pallas as pl
from jax.experimental.pallas import tpu as pltpu

_MARGIN = 0.1
_NNEG = 5
_TRIPLET = _NNEG + 2


def _body(nq, l, nchunk, chunk, nlabel,
          q_ref, rows_ref, trow_ref, qidx_ref, qidxs_ref, pidxf_ref, nnegf_ref,
          out_ref,
          pidx_s, nneg_s, hh_s, posval_s, negval_s, pmax_s, nmax_s, pcnt_s):
    c = pl.program_id(0)

    @pl.when(c == 0)
    def _init():
        qv = qidxs_ref[0:1, :]
        qi = qidx_ref[...]
        eq = qi == qv
        iota1 = lax.broadcasted_iota(jnp.int32, (nq, l), 1)
        qloc = jnp.min(jnp.where(eq, iota1, l), axis=1, keepdims=True)
        hh = qloc < l
        onehot = jnp.where((iota1 == qloc) & hh, 1.0, 0.0).astype(jnp.float32)
        pidx_s[...] = lax.dot(onehot, pidxf_ref[...],
                              precision=lax.Precision.HIGHEST,
                              preferred_element_type=jnp.float32).astype(jnp.int32)
        nneg_s[...] = lax.dot(onehot, nnegf_ref[...],
                              precision=lax.Precision.HIGHEST,
                              preferred_element_type=jnp.float32).astype(jnp.int32)
        hh_s[...] = hh.astype(jnp.int32)
        pmax_s[...] = jnp.full((nq, 1), -jnp.inf, jnp.float32)
        nmax_s[...] = jnp.full((nq, 1), -jnp.inf, jnp.float32)
        pcnt_s[...] = jnp.zeros((nq, 1), jnp.float32)

    rows = rows_ref[...]
    q = q_ref[...]
    q_hi = q.astype(jnp.bfloat16)
    q_lo = (q - q_hi.astype(jnp.float32)).astype(jnp.bfloat16)
    r_hi = rows.astype(jnp.bfloat16)
    r_lo = (rows - r_hi.astype(jnp.float32)).astype(jnp.bfloat16)
    dn = (((1,), (1,)), ((), ()))
    sim = (lax.dot_general(q_hi, r_hi, dn, preferred_element_type=jnp.float32)
           + (lax.dot_general(q_hi, r_lo, dn, preferred_element_type=jnp.float32)
              + lax.dot_general(q_lo, r_hi, dn,
                                preferred_element_type=jnp.float32)))
    trow = trow_ref[0]
    pidx = pidx_s[...]
    nneg = nneg_s[...]
    pos = trow == pidx[:, 0:1]
    negin = trow == nneg[:, 0:1]
    for j in range(1, nlabel):
        pos = pos | (trow == pidx[:, j:j + 1])
        negin = negin | (trow == nneg[:, j:j + 1])
    neg = ~negin

    posv = jnp.where(pos, sim, jnp.inf)
    negv = jnp.where(neg, sim, -jnp.inf)
    posval_s[:, pl.ds(c * chunk, chunk)] = posv
    negval_s[:, pl.ds(c * chunk, chunk)] = negv
    pmax_s[...] = jnp.maximum(
        pmax_s[...],
        jnp.max(jnp.where(pos, sim, -jnp.inf), axis=1, keepdims=True))
    nmax_s[...] = jnp.maximum(nmax_s[...], jnp.max(negv, axis=1, keepdims=True))
    pcnt_s[...] += jnp.sum(pos.astype(jnp.float32), axis=1, keepdims=True)

    @pl.when(c == nchunk - 1)
    def _finale():
        pmax = pmax_s[...]
        nmax = nmax_s[...]
        pcnt = pcnt_s[...]
        pt = nmax + _MARGIN
        nt = jnp.maximum(0.4, pmax) - _MARGIN
        zero = jnp.zeros((nq, 1), jnp.float32)
        pos_n = zero
        pos_sum = zero
        neg_n = zero
        neg_sum = zero
        for k in range(nchunk):
            pv = posval_s[:, k * chunk:(k + 1) * chunk]
            nv = negval_s[:, k * chunk:(k + 1) * chunk]
            selp = pv < pt
            seln = nv > nt
            pos_n = pos_n + jnp.sum(selp.astype(jnp.float32), axis=1, keepdims=True)
            pos_sum = pos_sum + jnp.sum(jnp.where(selp, 1.0 - pv, 0.0), axis=1,
                                        keepdims=True)
            neg_n = neg_n + jnp.sum(seln.astype(jnp.float32), axis=1, keepdims=True)
            neg_sum = neg_sum + jnp.sum(jnp.where(seln, nv, 0.0), axis=1,
                                        keepdims=True)
        pos_loss = jnp.where(pos_n > 0, pos_sum / jnp.maximum(pos_n, 1.0), 0.0)
        neg_loss = jnp.where(neg_n > 0, neg_sum / jnp.maximum(neg_n, 1.0), 0.0)
        contrib = jnp.where((hh_s[...] > 0) & (pcnt > 0), pos_loss + neg_loss, 0.0)
        out_ref[...] = (jnp.sum(contrib) / nq).reshape(1, 1)


@jax.jit
def kernel(inputs_col, targets_col, inputs_row, targets_row, qidxs, pidxs, nnegs):
    n, d = inputs_col.shape
    nrow = inputs_row.shape[0]
    l = qidxs.shape[0]
    nlabel = pidxs.shape[1]
    nq = n // _TRIPLET

    chunk = 4096
    nchunk = nrow // chunk

    q = inputs_col[::_TRIPLET]
    qidx = targets_col[::_TRIPLET].reshape(nq, 1)
    qidxs2 = jnp.broadcast_to(qidxs[None, :], (8, l))
    trow3 = targets_row.reshape(nchunk, 1, chunk)
    pidxf = pidxs.astype(jnp.float32)
    nnegf = nnegs.astype(jnp.float32)

    out = pl.pallas_call(
        functools.partial(_body, nq, l, nchunk, chunk, nlabel),
        grid=(nchunk,),
        in_specs=[
            pl.BlockSpec((nq, d), lambda c: (0, 0)),
            pl.BlockSpec((chunk, d), lambda c: (c, 0)),
            pl.BlockSpec((1, 1, chunk), lambda c: (c, 0, 0)),
            pl.BlockSpec((nq, 1), lambda c: (0, 0)),
            pl.BlockSpec((8, l), lambda c: (0, 0)),
            pl.BlockSpec((l, nlabel), lambda c: (0, 0)),
            pl.BlockSpec((l, nlabel), lambda c: (0, 0)),
        ],
        out_specs=pl.BlockSpec((1, 1), lambda c: (0, 0)),
        out_shape=jax.ShapeDtypeStruct((1, 1), jnp.float32),
        scratch_shapes=[
            pltpu.VMEM((nq, nlabel), jnp.int32),
            pltpu.VMEM((nq, nlabel), jnp.int32),
            pltpu.VMEM((nq, 1), jnp.int32),
            pltpu.VMEM((nq, nrow), jnp.float32),
            pltpu.VMEM((nq, nrow), jnp.float32),
            pltpu.VMEM((nq, 1), jnp.float32),
            pltpu.VMEM((nq, 1), jnp.float32),
            pltpu.VMEM((nq, 1), jnp.float32),
        ],
    )(q, inputs_row, trow3, qidx, qidxs2, pidxf, nnegf)
    return out.reshape(1)
